# baseline (device time: 114459 ns/iter reference)
import jax
import jax.numpy as jnp
from jax import lax
from jax.experimental import pallas as pl
from jax.experimental.pallas import tpu as pltpu

KQ = 8
D = 4
RA = 8
RX = 6


def kernel(x):
    m, n = x.shape
    qrows = m // 4
    ch = qrows // KQ
    h2 = ch // 2

    def body(
        x_any,
        out_any,
        commA,
        sendR,
        commY,
        commZ,
        commD,
        sendA,
        xf32,
        semA_send,
        semA_recv,
        semY_send,
        semY_recv,
        semZ_send,
        semZ_recv,
        semFY_send,
        semFY_recv,
        semFZ_send,
        semFZ_recv,
        xf32_sem,
        outO_sem,
        outY_sem,
        outZ_sem,
        outDY_sem,
        outDZ_sem,
    ):
        my_x = lax.axis_index("x")
        my_y = lax.axis_index("y")
        my_z = lax.axis_index("z")
        bz = my_z % 2
        xpeer = (1 - my_x, my_y, my_z)
        ypeer = (my_x, 1 - my_y, my_z)
        zpeer = (my_x, my_y, my_z + 1 - 2 * bz)

        barrier_sem = pltpu.get_barrier_semaphore()
        for p in (xpeer, ypeer, zpeer):
            pl.semaphore_signal(
                barrier_sem, inc=1, device_id=p,
                device_id_type=pl.DeviceIdType.MESH,
            )
        pl.semaphore_wait(barrier_sem, 3)

        def inner(yv, bv):
            qo = 2 * yv + bv
            qy = 2 * (1 - yv) + bv
            qz = 2 * yv + (1 - bv)
            qd = 2 * (1 - yv) + (1 - bv)

            def rdmaA(k):
                return pltpu.make_async_remote_copy(
                    src_ref=sendA.at[k % RA],
                    dst_ref=commA.at[pl.ds(k * ch, ch)],
                    send_sem=semA_send.at[k],
                    recv_sem=semA_recv.at[k],
                    device_id=xpeer,
                    device_id_type=pl.DeviceIdType.MESH,
                )

            def rdmaY(k):
                return pltpu.make_async_remote_copy(
                    src_ref=sendR.at[pl.ds(k * ch, ch)],
                    dst_ref=commY.at[pl.ds(k * ch, ch)],
                    send_sem=semY_send.at[k],
                    recv_sem=semY_recv.at[k],
                    device_id=ypeer,
                    device_id_type=pl.DeviceIdType.MESH,
                )

            def rdmaZ(k):
                return pltpu.make_async_remote_copy(
                    src_ref=sendR.at[pl.ds(k * ch, ch)],
                    dst_ref=commZ.at[pl.ds(k * ch, ch)],
                    send_sem=semZ_send.at[k],
                    recv_sem=semZ_recv.at[k],
                    device_id=zpeer,
                    device_id_type=pl.DeviceIdType.MESH,
                )

            def rdmaFY(k):
                return pltpu.make_async_remote_copy(
                    src_ref=commZ.at[pl.ds(k * ch, h2)],
                    dst_ref=commD.at[pl.ds(k * ch, h2)],
                    send_sem=semFY_send.at[k],
                    recv_sem=semFY_recv.at[k],
                    device_id=ypeer,
                    device_id_type=pl.DeviceIdType.MESH,
                )

            def rdmaFZ(k):
                return pltpu.make_async_remote_copy(
                    src_ref=commY.at[pl.ds(k * ch + h2, h2)],
                    dst_ref=commD.at[pl.ds(k * ch + h2, h2)],
                    send_sem=semFZ_send.at[k],
                    recv_sem=semFZ_recv.at[k],
                    device_id=zpeer,
                    device_id_type=pl.DeviceIdType.MESH,
                )

            def xf32_copy(k):
                return pltpu.make_async_copy(
                    x_any.at[pl.ds(qo * qrows + k * ch, ch)],
                    xf32.at[k % RX],
                    xf32_sem.at[k % RX],
                )

            def out_copy(src, src_off, qidx, dst_off, rows, sem):
                return pltpu.make_async_copy(
                    src.at[pl.ds(src_off, rows)],
                    out_any.at[pl.ds(qidx * qrows + dst_off, rows)],
                    sem,
                )

            def outO(k):
                return out_copy(sendR, k * ch, qo, k * ch, ch, outO_sem.at[k])

            def outY(k):
                return out_copy(commY, k * ch, qy, k * ch, ch, outY_sem.at[k])

            def outZ(k):
                return out_copy(commZ, k * ch, qz, k * ch, ch, outZ_sem.at[k])

            def outDY(k):
                return out_copy(commD, k * ch, qd, k * ch, h2, outDY_sem.at[k])

            def outDZ(k):
                return out_copy(
                    commD, k * ch + h2, qd, k * ch + h2, h2, outDZ_sem.at[k]
                )

            def feed(j):
                xf32_copy(j).wait()
                sendA[j % RA] = xf32[j % RX].astype(jnp.bfloat16)
                rdmaA(j).start()
                nxt = j + 2
                if RX <= nxt < KQ:
                    xf32_copy(nxt).start()

            for j in range(RX):
                xf32_copy(j).start()
            for j in range(D):
                feed(j)

            for k in range(KQ):
                if k + D < KQ:
                    feed(k + D)
                rdmaA(k).wait_recv()
                sendR[pl.ds(k * ch, ch), :] = (
                    sendA[k % RA].astype(jnp.float32)
                    + commA[pl.ds(k * ch, ch), :].astype(jnp.float32)
                ).astype(jnp.bfloat16)
                rdmaY(k).start()
                rdmaZ(k).start()
                outO(k).start()
                rdmaY(k).wait_recv()
                rdmaFZ(k).start()
                outY(k).start()
                rdmaZ(k).wait_recv()
                rdmaFY(k).start()
                outZ(k).start()
                rdmaFY(k).wait_recv()
                outDY(k).start()
                rdmaFZ(k).wait_recv()
                outDZ(k).start()

            for k in range(KQ):
                rdmaA(k).wait_send()
                rdmaY(k).wait_send()
                rdmaZ(k).wait_send()
                rdmaFY(k).wait_send()
                rdmaFZ(k).wait_send()
                outO(k).wait()
                outY(k).wait()
                outZ(k).wait()
                outDY(k).wait()
                outDZ(k).wait()

        for yv in (0, 1):
            for bv in (0, 1):
                @pl.when(jnp.logical_and(my_y == yv, bz == bv))
                def _(yv=yv, bv=bv):
                    inner(yv, bv)

    return pl.pallas_call(
        body,
        out_shape=jax.ShapeDtypeStruct((m, n), jnp.bfloat16),
        in_specs=[pl.BlockSpec(memory_space=pl.ANY)],
        out_specs=pl.BlockSpec(memory_space=pl.ANY),
        scratch_shapes=[
            pltpu.VMEM((qrows, n), jnp.bfloat16),
            pltpu.VMEM((qrows, n), jnp.bfloat16),
            pltpu.VMEM((qrows, n), jnp.bfloat16),
            pltpu.VMEM((qrows, n), jnp.bfloat16),
            pltpu.VMEM((qrows, n), jnp.bfloat16),
            pltpu.VMEM((RA, ch, n), jnp.bfloat16),
            pltpu.VMEM((RX, ch, n), jnp.float32),
            pltpu.SemaphoreType.DMA((KQ,)),
            pltpu.SemaphoreType.DMA((KQ,)),
            pltpu.SemaphoreType.DMA((KQ,)),
            pltpu.SemaphoreType.DMA((KQ,)),
            pltpu.SemaphoreType.DMA((KQ,)),
            pltpu.SemaphoreType.DMA((KQ,)),
            pltpu.SemaphoreType.DMA((KQ,)),
            pltpu.SemaphoreType.DMA((KQ,)),
            pltpu.SemaphoreType.DMA((KQ,)),
            pltpu.SemaphoreType.DMA((KQ,)),
            pltpu.SemaphoreType.DMA((RX,)),
            pltpu.SemaphoreType.DMA((KQ,)),
            pltpu.SemaphoreType.DMA((KQ,)),
            pltpu.SemaphoreType.DMA((KQ,)),
            pltpu.SemaphoreType.DMA((KQ,)),
            pltpu.SemaphoreType.DMA((KQ,)),
        ],
        compiler_params=pltpu.CompilerParams(collective_id=0),
    )(x)


# device time: 87166 ns/iter; 1.3131x vs baseline; 1.3131x over previous
import jax
import jax.numpy as jnp
from jax import lax
from jax.experimental import pallas as pl
from jax.experimental.pallas import tpu as pltpu

KQ = 8
D = 4
RA = 8
RX = 6


def kernel(x):
    m, n = x.shape
    qrows = m // 4
    ch = qrows // KQ
    h2 = ch // 2

    def body(
        x_any,
        out_any,
        commA,
        sendR,
        commY,
        commZ,
        commD,
        sendA,
        xf32,
        semA_send,
        semA_recv,
        semY_send,
        semY_recv,
        semZ_send,
        semZ_recv,
        semFY_send,
        semFY_recv,
        semFZ_send,
        semFZ_recv,
        xf32_sem,
        outO_sem,
        outY_sem,
        outZ_sem,
        outDY_sem,
        outDZ_sem,
    ):
        my_x = lax.axis_index("x")
        my_y = lax.axis_index("y")
        my_z = lax.axis_index("z")
        bz = my_z % 2
        xpeer = (1 - my_x, my_y, my_z)
        ypeer = (my_x, 1 - my_y, my_z)
        zpeer = (my_x, my_y, my_z + 1 - 2 * bz)

        barrier_sem = pltpu.get_barrier_semaphore()
        for p in (xpeer, ypeer, zpeer):
            pl.semaphore_signal(
                barrier_sem, inc=1, device_id=p,
                device_id_type=pl.DeviceIdType.MESH,
            )
        pl.semaphore_wait(barrier_sem, 3)

        def inner(yv, bv):
            qo = 2 * yv + bv
            qy = 2 * (1 - yv) + bv
            qz = 2 * yv + (1 - bv)
            qd = 2 * (1 - yv) + (1 - bv)

            def rdmaA(k):
                return pltpu.make_async_remote_copy(
                    src_ref=sendA.at[k % RA],
                    dst_ref=commA.at[pl.ds(k * ch, ch)],
                    send_sem=semA_send.at[k],
                    recv_sem=semA_recv.at[k],
                    device_id=xpeer,
                    device_id_type=pl.DeviceIdType.MESH,
                )

            def rdmaY(k):
                return pltpu.make_async_remote_copy(
                    src_ref=sendR.at[pl.ds(k * ch, ch)],
                    dst_ref=commY.at[pl.ds(k * ch, ch)],
                    send_sem=semY_send.at[k],
                    recv_sem=semY_recv.at[k],
                    device_id=ypeer,
                    device_id_type=pl.DeviceIdType.MESH,
                )

            def rdmaZ(k):
                return pltpu.make_async_remote_copy(
                    src_ref=sendR.at[pl.ds(k * ch, ch)],
                    dst_ref=commZ.at[pl.ds(k * ch, ch)],
                    send_sem=semZ_send.at[k],
                    recv_sem=semZ_recv.at[k],
                    device_id=zpeer,
                    device_id_type=pl.DeviceIdType.MESH,
                )

            def rdmaFY(k):
                return pltpu.make_async_remote_copy(
                    src_ref=commZ.at[pl.ds(k * ch, h2)],
                    dst_ref=commD.at[pl.ds(k * ch, h2)],
                    send_sem=semFY_send.at[k],
                    recv_sem=semFY_recv.at[k],
                    device_id=ypeer,
                    device_id_type=pl.DeviceIdType.MESH,
                )

            def rdmaFZ(k):
                return pltpu.make_async_remote_copy(
                    src_ref=commY.at[pl.ds(k * ch + h2, h2)],
                    dst_ref=commD.at[pl.ds(k * ch + h2, h2)],
                    send_sem=semFZ_send.at[k],
                    recv_sem=semFZ_recv.at[k],
                    device_id=zpeer,
                    device_id_type=pl.DeviceIdType.MESH,
                )

            def xf32_copy(k):
                return pltpu.make_async_copy(
                    x_any.at[pl.ds(qo * qrows + k * ch, ch)],
                    xf32.at[k % RX],
                    xf32_sem.at[k % RX],
                )

            def out_copy(src, src_off, qidx, dst_off, rows, sem):
                return pltpu.make_async_copy(
                    src.at[pl.ds(src_off, rows)],
                    out_any.at[pl.ds(qidx * qrows + dst_off, rows)],
                    sem,
                )

            def outO(k):
                return out_copy(sendR, k * ch, qo, k * ch, ch, outO_sem.at[k])

            def outY(k):
                return out_copy(commY, k * ch, qy, k * ch, ch, outY_sem.at[k])

            def outZ(k):
                return out_copy(commZ, k * ch, qz, k * ch, ch, outZ_sem.at[k])

            def outDY(k):
                return out_copy(commD, k * ch, qd, k * ch, h2, outDY_sem.at[k])

            def outDZ(k):
                return out_copy(
                    commD, k * ch + h2, qd, k * ch + h2, h2, outDZ_sem.at[k]
                )

            def feed(j):
                xf32_copy(j).wait()
                sendA[j % RA] = xf32[j % RX].astype(jnp.bfloat16)
                rdmaA(j).start()
                nxt = j + 2
                if RX <= nxt < KQ:
                    xf32_copy(nxt).start()

            for j in range(RX):
                xf32_copy(j).start()
            for j in range(D):
                feed(j)

            def consumeYZ(k):
                rdmaY(k).wait_recv()
                rdmaFZ(k).start()
                outY(k).start()
                rdmaZ(k).wait_recv()
                rdmaFY(k).start()
                outZ(k).start()

            def consumeD(k):
                rdmaFY(k).wait_recv()
                outDY(k).start()
                rdmaFZ(k).wait_recv()
                outDZ(k).start()

            for k in range(KQ):
                if k + D < KQ:
                    feed(k + D)
                rdmaA(k).wait_recv()
                sendR[pl.ds(k * ch, ch), :] = (
                    sendA[k % RA].astype(jnp.float32)
                    + commA[pl.ds(k * ch, ch), :].astype(jnp.float32)
                ).astype(jnp.bfloat16)
                rdmaY(k).start()
                rdmaZ(k).start()
                outO(k).start()
                if k >= 1:
                    consumeYZ(k - 1)
                if k >= 2:
                    consumeD(k - 2)
            consumeYZ(KQ - 1)
            consumeD(KQ - 2)
            consumeD(KQ - 1)

            for k in range(KQ):
                rdmaA(k).wait_send()
                rdmaY(k).wait_send()
                rdmaZ(k).wait_send()
                rdmaFY(k).wait_send()
                rdmaFZ(k).wait_send()
                outO(k).wait()
                outY(k).wait()
                outZ(k).wait()
                outDY(k).wait()
                outDZ(k).wait()

        for yv in (0, 1):
            for bv in (0, 1):
                @pl.when(jnp.logical_and(my_y == yv, bz == bv))
                def _(yv=yv, bv=bv):
                    inner(yv, bv)

    return pl.pallas_call(
        body,
        out_shape=jax.ShapeDtypeStruct((m, n), jnp.bfloat16),
        in_specs=[pl.BlockSpec(memory_space=pl.ANY)],
        out_specs=pl.BlockSpec(memory_space=pl.ANY),
        scratch_shapes=[
            pltpu.VMEM((qrows, n), jnp.bfloat16),
            pltpu.VMEM((qrows, n), jnp.bfloat16),
            pltpu.VMEM((qrows, n), jnp.bfloat16),
            pltpu.VMEM((qrows, n), jnp.bfloat16),
            pltpu.VMEM((qrows, n), jnp.bfloat16),
            pltpu.VMEM((RA, ch, n), jnp.bfloat16),
            pltpu.VMEM((RX, ch, n), jnp.float32),
            pltpu.SemaphoreType.DMA((KQ,)),
            pltpu.SemaphoreType.DMA((KQ,)),
            pltpu.SemaphoreType.DMA((KQ,)),
            pltpu.SemaphoreType.DMA((KQ,)),
            pltpu.SemaphoreType.DMA((KQ,)),
            pltpu.SemaphoreType.DMA((KQ,)),
            pltpu.SemaphoreType.DMA((KQ,)),
            pltpu.SemaphoreType.DMA((KQ,)),
            pltpu.SemaphoreType.DMA((KQ,)),
            pltpu.SemaphoreType.DMA((KQ,)),
            pltpu.SemaphoreType.DMA((RX,)),
            pltpu.SemaphoreType.DMA((KQ,)),
            pltpu.SemaphoreType.DMA((KQ,)),
            pltpu.SemaphoreType.DMA((KQ,)),
            pltpu.SemaphoreType.DMA((KQ,)),
            pltpu.SemaphoreType.DMA((KQ,)),
        ],
        compiler_params=pltpu.CompilerParams(collective_id=0),
    )(x)
